# baseline (speedup 1.0000x reference)
import functools

import jax
import jax.numpy as jnp
from jax import lax
from jax.experimental import pallas as pl
from jax.experimental.pallas import tpu as pltpu
from jax.experimental.pallas import tpu_sc as plsc

VOCAB = 100000
DIM = 64
B = 1024
SENT_COUNT = 10
SENT_LENGTH = 20

N_UR = B * SENT_COUNT * SENT_LENGTH
N_UI = B * SENT_LENGTH

NC = 2
NS = 16
NW = NC * NS

CHUNK = 128
K = 5
GROUP = K * CHUNK
HG = GROUP // 2

CH_UR = N_UR // NW // CHUNK
CH_UI = N_UI // NW // CHUNK
NG_UR = CH_UR // K

PW_UR = N_UR // NW
PW_UI = N_UI // NW


def _emb_packed(idx_hbm, table, out,
                idx_v, rows_v, gsem0, gsem1, ssem0, ssem1):
    wid = lax.axis_index("s") * NC + lax.axis_index("c")
    gsems = (gsem0, gsem1)
    ssems = (ssem0, ssem1)

    pltpu.sync_copy(idx_hbm.at[pl.ds(wid * CH_UR, CH_UR)], idx_v)

    def fire_group(g, p):
        for b in range(K):
            pltpu.async_copy(
                table.at[idx_v.at[g * K + b]],
                rows_v.at[p, pl.ds(b * CHUNK, CHUNK)],
                gsems[p])

    def drain_gathers(p):
        pltpu.make_async_copy(table.at[pl.ds(0, GROUP)],
                              rows_v.at[p], gsems[p]).wait()

    def store_group(g, p):
        base = wid * (PW_UR // 2) + g * HG
        s1 = pltpu.async_copy(rows_v.at[p, pl.ds(0, HG)],
                              out.at[pl.ds(base, HG), pl.ds(0, DIM)],
                              ssems[p])
        s2 = pltpu.async_copy(rows_v.at[p, pl.ds(HG, HG)],
                              out.at[pl.ds(base, HG), pl.ds(DIM, DIM)],
                              ssems[p])
        s1.wait()
        s2.wait()

    fire_group(0, 0)
    fire_group(1, 1)

    def body(i, carry):
        for p in (0, 1):
            g = 2 * i + p
            drain_gathers(p)
            store_group(g, p)
            fire_group(g + 2, p)
        return carry

    lax.fori_loop(0, NG_UR // 2 - 1, body, 0)
    for p in (0, 1):
        g = NG_UR - 2 + p
        drain_gathers(p)
        store_group(g, p)


def _emb_ui(idx_hbm, table, out_ui,
            idx_v, rows_v, gsem0, gsem1, ssem0, ssem1):
    wid = lax.axis_index("s") * NC + lax.axis_index("c")
    pltpu.sync_copy(idx_hbm.at[pl.ds(wid * CH_UI, CH_UI)], idx_v)
    for b in range(K):
        pltpu.async_copy(
            table.at[idx_v.at[b]],
            rows_v.at[0, pl.ds(b * CHUNK, CHUNK)],
            gsem0)
    pltpu.make_async_copy(table.at[pl.ds(0, GROUP)],
                          rows_v.at[0], gsem0).wait()
    pltpu.sync_copy(rows_v.at[0], out_ui.at[pl.ds(wid * PW_UI, PW_UI)])


def _unpack_kernel(x_ref, o_ref):
    x = x_ref[...]
    a = x[:, None, :DIM]
    b = x[:, None, DIM:]
    o_ref[...] = jnp.concatenate([a, b], axis=1).reshape(2 * x.shape[0], DIM)


def _unpack(packed):
    n2 = packed.shape[0]
    blk = 800
    return pl.pallas_call(
        _unpack_kernel,
        grid=(n2 // blk,),
        in_specs=[pl.BlockSpec((blk, 2 * DIM), lambda i: (i, 0))],
        out_specs=pl.BlockSpec((2 * blk, DIM), lambda i: (i, 0)),
        out_shape=jax.ShapeDtypeStruct((2 * n2, DIM), jnp.float32),
    )(packed)


def _permute(idx2d):
    flat = idx2d.reshape(-1, HG, 2)
    return flat.transpose(0, 2, 1).reshape(-1, CHUNK)


@jax.jit
def _run(idx_ur, idx_ir, idx_ui, word_emb):
    mesh = plsc.VectorSubcoreMesh(core_axis_name="c", subcore_axis_name="s")
    common = dict(
        mesh=mesh,
        compiler_params=pltpu.CompilerParams(use_tc_tiling_on_sc=False),
    )
    big_scratch = [
        pltpu.VMEM((CH_UR, CHUNK), jnp.int32),
        pltpu.VMEM((2, GROUP, DIM), jnp.float32),
        pltpu.SemaphoreType.DMA,
        pltpu.SemaphoreType.DMA,
        pltpu.SemaphoreType.DMA,
        pltpu.SemaphoreType.DMA,
    ]
    packed_type = jax.ShapeDtypeStruct((N_UR // 2, 2 * DIM), jnp.float32)
    pk_ur = pl.kernel(_emb_packed, out_type=packed_type,
                      scratch_types=big_scratch, **common)(idx_ur, word_emb)
    out_ur = _unpack(pk_ur)
    pk_ir = pl.kernel(_emb_packed, out_type=packed_type,
                      scratch_types=big_scratch, **common)(idx_ir, word_emb)
    out_ir = _unpack(pk_ir)
    out_ui = pl.kernel(
        _emb_ui,
        out_type=jax.ShapeDtypeStruct((N_UI, DIM), jnp.float32),
        scratch_types=[
            pltpu.VMEM((CH_UI, CHUNK), jnp.int32),
            pltpu.VMEM((2, GROUP, DIM), jnp.float32),
            pltpu.SemaphoreType.DMA,
            pltpu.SemaphoreType.DMA,
            pltpu.SemaphoreType.DMA,
            pltpu.SemaphoreType.DMA,
        ],
        **common,
    )(idx_ui, word_emb)
    return out_ur, out_ir, out_ui


def kernel(user_reviews, item_reviews, ui_review, word_emb):
    idx_ur = _permute(user_reviews.reshape(-1, CHUNK))
    idx_ir = _permute(item_reviews.reshape(-1, CHUNK))
    idx_ui = ui_review.reshape(-1, CHUNK)
    out_ur, out_ir, out_ui = _run(idx_ur, idx_ir, idx_ui, word_emb)
    return (
        out_ur.reshape(B, SENT_COUNT * SENT_LENGTH, DIM),
        out_ir.reshape(B, SENT_COUNT * SENT_LENGTH, DIM),
        out_ui.reshape(B, SENT_LENGTH, DIM),
    )


# speedup vs baseline: 1.5556x; 1.5556x over previous
import functools

import jax
import jax.numpy as jnp
from jax import lax
from jax.experimental import pallas as pl
from jax.experimental.pallas import tpu as pltpu
from jax.experimental.pallas import tpu_sc as plsc

VOCAB = 100000
DIM = 64
B = 1024
SENT_COUNT = 10
SENT_LENGTH = 20

N_UR = B * SENT_COUNT * SENT_LENGTH
N_UI = B * SENT_LENGTH

NC = 2
NS = 16
NW = NC * NS

CHUNK = 128
K = 5
GROUP = K * CHUNK

CH_UR = N_UR // NW // CHUNK
CH_UI = N_UI // NW // CHUNK
NG_UR = CH_UR // K

PW_UR = N_UR // NW
PW_UI = N_UI // NW


def _pipeline(table, idx_v, rows_v, gsems, ssems, wid, stores, ng):
    def fire_group(g, p):
        for b in range(K):
            pltpu.async_copy(
                table.at[idx_v.at[g * K + b]],
                rows_v.at[p, pl.ds(b * CHUNK, CHUNK)],
                gsems[p])

    def drain_gathers(p):
        pltpu.make_async_copy(table.at[pl.ds(0, GROUP)],
                              rows_v.at[p], gsems[p]).wait()

    fire_group(0, 0)
    if ng > 1:
        fire_group(1, 1)

    def body(i, carry):
        for p in (0, 1):
            g = 2 * i + p
            drain_gathers(p)
            stores(g, p)
            fire_group(g + 2, p)
        return carry

    lax.fori_loop(0, (ng - 2) // 2, body, 0)
    for p in (0, 1):
        g = ng - 2 + p
        drain_gathers(p)
        stores(g, p)
    return drain_gathers


def _emb_ur(idx_hbm, table, out_ur,
            idx_v, rows_v, gsem0, gsem1, ssem0, ssem1):
    wid = lax.axis_index("s") * NC + lax.axis_index("c")
    pltpu.sync_copy(idx_hbm.at[pl.ds(wid * CH_UR, CH_UR)], idx_v)

    def stores(g, p):
        pltpu.async_copy(
            rows_v.at[p],
            out_ur.at[pl.ds(wid * PW_UR + g * GROUP, GROUP)],
            (ssem0, ssem1)[p]).wait()

    _pipeline(table, idx_v, rows_v, (gsem0, gsem1), (ssem0, ssem1),
              wid, stores, NG_UR)


def _emb_ui(idx_hbm, table, out_ui,
            idx_v, rows_v, gsem0, gsem1, ssem0, ssem1):
    wid = lax.axis_index("s") * NC + lax.axis_index("c")
    pltpu.sync_copy(idx_hbm.at[pl.ds(wid * CH_UI, CH_UI)], idx_v)
    for b in range(K):
        pltpu.async_copy(
            table.at[idx_v.at[b]],
            rows_v.at[0, pl.ds(b * CHUNK, CHUNK)],
            gsem0)
    pltpu.make_async_copy(table.at[pl.ds(0, GROUP)],
                          rows_v.at[0], gsem0).wait()
    pltpu.sync_copy(rows_v.at[0], out_ui.at[pl.ds(wid * PW_UI, PW_UI)])


@jax.jit
def _run(idx_ur, idx_ir, idx_ui, word_emb):
    mesh = plsc.VectorSubcoreMesh(core_axis_name="c", subcore_axis_name="s")
    common = dict(
        mesh=mesh,
        compiler_params=pltpu.CompilerParams(use_tc_tiling_on_sc=False),
    )
    big_scratch = [
        pltpu.VMEM((CH_UR, CHUNK), jnp.int32),
        pltpu.VMEM((2, GROUP, DIM), jnp.float32),
        pltpu.SemaphoreType.DMA,
        pltpu.SemaphoreType.DMA,
        pltpu.SemaphoreType.DMA,
        pltpu.SemaphoreType.DMA,
    ]
    out_ur = pl.kernel(
        _emb_ur,
        out_type=jax.ShapeDtypeStruct((N_UR, DIM), jnp.float32),
        scratch_types=big_scratch,
        **common,
    )(idx_ur, word_emb)
    out_ir = pl.kernel(
        _emb_ur,
        out_type=jax.ShapeDtypeStruct((N_UR, DIM), jnp.float32),
        scratch_types=big_scratch,
        **common,
    )(idx_ir, word_emb)
    out_ui = pl.kernel(
        _emb_ui,
        out_type=jax.ShapeDtypeStruct((N_UI, DIM), jnp.float32),
        scratch_types=[
            pltpu.VMEM((CH_UI, CHUNK), jnp.int32),
            pltpu.VMEM((2, GROUP, DIM), jnp.float32),
            pltpu.SemaphoreType.DMA,
            pltpu.SemaphoreType.DMA,
            pltpu.SemaphoreType.DMA,
            pltpu.SemaphoreType.DMA,
        ],
        **common,
    )(idx_ui, word_emb)
    return out_ur, out_ir, out_ui


def kernel(user_reviews, item_reviews, ui_review, word_emb):
    idx_ur = user_reviews.reshape(-1, CHUNK)
    idx_ir = item_reviews.reshape(-1, CHUNK)
    idx_ui = ui_review.reshape(-1, CHUNK)
    out_ur, out_ir, out_ui = _run(idx_ur, idx_ir, idx_ui, word_emb)
    return (
        out_ur.reshape(B, SENT_COUNT * SENT_LENGTH, DIM),
        out_ir.reshape(B, SENT_COUNT * SENT_LENGTH, DIM),
        out_ui.reshape(B, SENT_LENGTH, DIM),
    )
